# Initial kernel scaffold; baseline (speedup 1.0000x reference)
import jax
import jax.numpy as jnp
from jax.experimental import pallas as pl


def kernel(x, adj, W_ks, a_k_param, W_o, a_att, a_out):
    raise NotImplementedError("write your pallas kernel here")


# speedup vs baseline: 1.0301x; 1.0301x over previous
import functools
import jax
import jax.numpy as jnp
from jax import lax
from jax.experimental import pallas as pl

N_NODES = 10000
N_FEAT = 256
N_HID = 64
N_FACT = 4
N_CLASS = 16
N_EDGES = 160000
ALPHA = 0.2
EPS = 1e-16

ROW_BLK = 1000


def _lrelu(z):
    return jnp.maximum(z, ALPHA * z)


def _elu(x):
    return jnp.where(x > 0, x, jnp.exp(jnp.minimum(x, 0.0)) - 1.0)


def _bdot(a, b):
    return jnp.dot(a.astype(jnp.bfloat16), b.astype(jnp.bfloat16),
                   preferred_element_type=jnp.float32)


def _tc1_body(x_ref, wc_ref, acat_ref, h_ref, nl_ref):
    xb = x_ref[...]
    hb = _bdot(xb, wc_ref[...])
    h_ref[...] = hb
    nl_ref[...] = _bdot(hb, acat_ref[...])


def _tc1(x, wc, acat):
    grid = N_NODES // ROW_BLK
    return pl.pallas_call(
        _tc1_body,
        grid=(grid,),
        in_specs=[
            pl.BlockSpec((ROW_BLK, N_FEAT), lambda i: (i, 0)),
            pl.BlockSpec((N_FEAT, N_FEAT), lambda i: (0, 0)),
            pl.BlockSpec((N_FEAT, 16), lambda i: (0, 0)),
        ],
        out_specs=[
            pl.BlockSpec((ROW_BLK, N_FEAT), lambda i: (i, 0)),
            pl.BlockSpec((ROW_BLK, 16), lambda i: (i, 0)),
        ],
        out_shape=[
            jax.ShapeDtypeStruct((N_NODES, N_FEAT), jnp.float32),
            jax.ShapeDtypeStruct((N_NODES, 16), jnp.float32),
        ],
    )(x, wc, acat)


def _tc2_body(acc_ref, rs_ref, wo_ref, ao_ref, xo_ref, uv_ref):
    acc = acc_ref[...]
    rs = rs_ref[...]
    cols = []
    for k in range(N_FACT):
        hp = acc[:, k * N_HID:(k + 1) * N_HID] / (rs[:, k:k + 1] + EPS)
        cols.append(_elu(hp))
    xc = jnp.concatenate(cols, axis=1)
    xo = _bdot(xc, wo_ref[...])
    xo_ref[...] = xo
    uv_ref[...] = _bdot(xo, ao_ref[...])


def _tc2(acc, rs, wo, ao):
    grid = N_NODES // ROW_BLK
    return pl.pallas_call(
        _tc2_body,
        grid=(grid,),
        in_specs=[
            pl.BlockSpec((ROW_BLK, N_FEAT), lambda i: (i, 0)),
            pl.BlockSpec((ROW_BLK, 8), lambda i: (i, 0)),
            pl.BlockSpec((N_FEAT, N_CLASS), lambda i: (0, 0)),
            pl.BlockSpec((N_CLASS, 8), lambda i: (0, 0)),
        ],
        out_specs=[
            pl.BlockSpec((ROW_BLK, N_CLASS), lambda i: (i, 0)),
            pl.BlockSpec((ROW_BLK, 8), lambda i: (i, 0)),
        ],
        out_shape=[
            jax.ShapeDtypeStruct((N_NODES, N_CLASS), jnp.float32),
            jax.ShapeDtypeStruct((N_NODES, 8), jnp.float32),
        ],
    )(acc, rs, wo, ao)


def _tc3_body(acc_ref, rs_ref, out_ref):
    xf = _elu(acc_ref[...] / (rs_ref[:, 0:1] + EPS))
    m = jnp.max(xf, axis=1, keepdims=True)
    s = xf - m
    out_ref[...] = s - jnp.log(jnp.sum(jnp.exp(s), axis=1, keepdims=True))


def _tc3(acc2, rs2):
    grid = N_NODES // ROW_BLK
    return pl.pallas_call(
        _tc3_body,
        grid=(grid,),
        in_specs=[
            pl.BlockSpec((ROW_BLK, N_CLASS), lambda i: (i, 0)),
            pl.BlockSpec((ROW_BLK, 8), lambda i: (i, 0)),
        ],
        out_specs=pl.BlockSpec((ROW_BLK, N_CLASS), lambda i: (i, 0)),
        out_shape=jax.ShapeDtypeStruct((N_NODES, N_CLASS), jnp.float32),
    )(acc2, rs2)


def kernel(x, adj, W_ks, a_k_param, W_o, a_att, a_out):
    wc = jnp.transpose(W_ks, (1, 0, 2)).reshape(N_FEAT, N_FACT * N_HID)
    a_s = jnp.transpose(a_k_param[:, :N_FEAT])
    a_d = jnp.transpose(a_k_param[:, N_FEAT:])
    a_p = jnp.zeros((N_FEAT, N_FACT), jnp.float32)
    a_q = jnp.zeros((N_FEAT, N_FACT), jnp.float32)
    for k in range(N_FACT):
        a_p = a_p.at[k * N_HID:(k + 1) * N_HID, k].set(a_att[k, 0, :N_HID])
        a_q = a_q.at[k * N_HID:(k + 1) * N_HID, k].set(a_att[k, 0, N_HID:])
    acat = jnp.concatenate([a_s, a_d, a_p, a_q], axis=1)
    ao = jnp.zeros((N_CLASS, 8), jnp.float32)
    ao = ao.at[:, 0].set(a_out[0, :N_CLASS])
    ao = ao.at[:, 1].set(a_out[0, N_CLASS:])

    h, nl = _tc1(x, wc, acat)
    S, D, P, Q = nl[:, 0:4], nl[:, 4:8], nl[:, 8:12], nl[:, 12:16]

    src, dst = jnp.nonzero(adj, size=N_EDGES, fill_value=0)
    num_edges = jnp.count_nonzero(adj)
    valid = (jnp.arange(N_EDGES) < num_edges)
    validf = valid.astype(jnp.float32)

    l = S[src] + D[dst]
    wins = jnp.zeros((N_EDGES, N_FACT), jnp.float32)
    for i in range(N_FACT):
        for j in range(i + 1, N_FACT):
            b = (l[:, i] >= l[:, j]).astype(jnp.float32)
            wins = wins.at[:, i].add(b)
            wins = wins.at[:, j].add(1.0 - b)
    mask = (wins >= 2.0).astype(jnp.float32) * validf[:, None]
    ex = jnp.exp(l - jnp.max(l, axis=1, keepdims=True))
    sm = ex / jnp.sum(ex, axis=1, keepdims=True)
    top2sum = jnp.sum(mask * sm, axis=1)
    att_loss = jnp.sum(validf * (1.0 - top2sum)) / jnp.sum(validf)

    z = P[src] + Q[dst]
    e1 = jnp.exp(-_lrelu(z)) * mask
    rowsum = jax.ops.segment_sum(e1, src, num_segments=N_NODES)
    w_full = jnp.repeat(e1, N_HID, axis=1)
    acc = jax.ops.segment_sum(w_full * h[dst], src, num_segments=N_NODES)

    rs_pad = jnp.zeros((N_NODES, 8), jnp.float32).at[:, :N_FACT].set(rowsum)
    xo, uv = _tc2(acc, rs_pad, W_o, ao)
    U, V = uv[:, 0], uv[:, 1]

    e2 = jnp.exp(-_lrelu(U[src] + V[dst])) * validf
    rowsum2 = jax.ops.segment_sum(e2, src, num_segments=N_NODES)
    acc2 = jax.ops.segment_sum(e2[:, None] * xo[dst], src, num_segments=N_NODES)

    rs2_pad = jnp.zeros((N_NODES, 8), jnp.float32).at[:, 0].set(rowsum2)
    logp = _tc3(acc2, rs2_pad)
    return (logp, att_loss)


# speedup vs baseline: 8.8401x; 8.5814x over previous
import functools
import jax
import jax.numpy as jnp
from jax import lax
from jax.experimental import pallas as pl

N_NODES = 10000
N_FEAT = 256
N_HID = 64
N_FACT = 4
N_CLASS = 16
N_EDGES = 160000
ALPHA = 0.2
EPS = 1e-16

ROW_BLK = 1000


def _lrelu(z):
    return jnp.maximum(z, ALPHA * z)


def _elu(x):
    return jnp.where(x > 0, x, jnp.exp(jnp.minimum(x, 0.0)) - 1.0)


def _bdot(a, b):
    return jnp.dot(a.astype(jnp.bfloat16), b.astype(jnp.bfloat16),
                   preferred_element_type=jnp.float32)


def _tc1_body(x_ref, wc_ref, acat_ref, h_ref, nl_ref):
    xb = x_ref[...]
    hb = _bdot(xb, wc_ref[...])
    h_ref[...] = hb
    nl_ref[...] = _bdot(hb, acat_ref[...])


def _tc1(x, wc, acat):
    grid = N_NODES // ROW_BLK
    return pl.pallas_call(
        _tc1_body,
        grid=(grid,),
        in_specs=[
            pl.BlockSpec((ROW_BLK, N_FEAT), lambda i: (i, 0)),
            pl.BlockSpec((N_FEAT, N_FEAT), lambda i: (0, 0)),
            pl.BlockSpec((N_FEAT, 16), lambda i: (0, 0)),
        ],
        out_specs=[
            pl.BlockSpec((ROW_BLK, N_FEAT), lambda i: (i, 0)),
            pl.BlockSpec((ROW_BLK, 16), lambda i: (i, 0)),
        ],
        out_shape=[
            jax.ShapeDtypeStruct((N_NODES, N_FEAT), jnp.float32),
            jax.ShapeDtypeStruct((N_NODES, 16), jnp.float32),
        ],
    )(x, wc, acat)


def _tc2_body(acc_ref, rs_ref, wo_ref, ao_ref, xo_ref, uv_ref, sc_ref):
    i = pl.program_id(0)
    acc = acc_ref[...]
    rs = rs_ref[...]
    cols = []
    for k in range(N_FACT):
        hp = acc[:, k * N_HID:(k + 1) * N_HID] / (rs[:, k:k + 1] + EPS)
        cols.append(_elu(hp))
    xc = jnp.concatenate(cols, axis=1)
    xo = _bdot(xc, wo_ref[...])
    xo_ref[...] = xo
    uv_ref[...] = _bdot(xo, ao_ref[...])
    upd = jnp.concatenate(
        [jnp.sum(rs[:, 4:5], axis=0, keepdims=True),
         jnp.sum(rs[:, 5:6], axis=0, keepdims=True),
         jnp.zeros((1, 6), jnp.float32)], axis=1)

    @pl.when(i == 0)
    def _():
        sc_ref[...] = upd

    @pl.when(i != 0)
    def _():
        sc_ref[...] = sc_ref[...] + upd


def _tc2(acc, rs, wo, ao):
    grid = N_NODES // ROW_BLK
    return pl.pallas_call(
        _tc2_body,
        grid=(grid,),
        in_specs=[
            pl.BlockSpec((ROW_BLK, N_FEAT), lambda i: (i, 0)),
            pl.BlockSpec((ROW_BLK, 8), lambda i: (i, 0)),
            pl.BlockSpec((N_FEAT, N_CLASS), lambda i: (0, 0)),
            pl.BlockSpec((N_CLASS, 8), lambda i: (0, 0)),
        ],
        out_specs=[
            pl.BlockSpec((ROW_BLK, N_CLASS), lambda i: (i, 0)),
            pl.BlockSpec((ROW_BLK, 8), lambda i: (i, 0)),
            pl.BlockSpec((1, 8), lambda i: (0, 0)),
        ],
        out_shape=[
            jax.ShapeDtypeStruct((N_NODES, N_CLASS), jnp.float32),
            jax.ShapeDtypeStruct((N_NODES, 8), jnp.float32),
            jax.ShapeDtypeStruct((1, 8), jnp.float32),
        ],
    )(acc, rs, wo, ao)


SBLK = 80


def _passA_body(adj_ref, nls_ref, nlT_ref, h_ref, acc_ref, rs_ref):
    a = adj_ref[...]
    Sn = nls_ref[...]
    Tn = nlT_ref[...]
    l = [Sn[:, k:k + 1] + Tn[4 + k:5 + k, :] for k in range(N_FACT)]
    zero = jnp.zeros_like(l[0])
    wins = [zero, zero, zero, zero]
    for i in range(N_FACT):
        for jj in range(i + 1, N_FACT):
            b = (l[i] >= l[jj]).astype(jnp.float32)
            wins[i] = wins[i] + b
            wins[jj] = wins[jj] + (1.0 - b)
    masks = [w >= 2.0 for w in wins]
    m = jnp.maximum(jnp.maximum(l[0], l[1]), jnp.maximum(l[2], l[3]))
    ex = [jnp.exp(lk - m) for lk in l]
    sumex = ex[0] + ex[1] + ex[2] + ex[3]
    topex = sum(jnp.where(mk, ek, 0.0) for mk, ek in zip(masks, ex))
    lossrow = jnp.sum(a * (1.0 - topex / sumex), axis=1, keepdims=True)
    countrow = jnp.sum(a, axis=1, keepdims=True)
    mats = []
    rows = []
    for k in range(N_FACT):
        z = Sn[:, 8 + k:9 + k] + Tn[12 + k:13 + k, :]
        w = jnp.where(masks[k], jnp.exp(-_lrelu(z)), 0.0) * a
        mats.append(jnp.dot(w, h_ref[:, k * N_HID:(k + 1) * N_HID],
                            preferred_element_type=jnp.float32,
                            precision=lax.Precision.HIGHEST))
        rows.append(jnp.sum(w, axis=1, keepdims=True))
    acc_ref[...] = jnp.concatenate(mats, axis=1)
    rs_ref[...] = jnp.concatenate(
        rows + [lossrow, countrow, jnp.zeros((SBLK, 2), jnp.float32)], axis=1)


def _passA(adj, nl, nlT, h):
    return pl.pallas_call(
        _passA_body,
        grid=(N_NODES // SBLK,),
        in_specs=[
            pl.BlockSpec((SBLK, N_NODES), lambda i: (i, 0)),
            pl.BlockSpec((SBLK, 16), lambda i: (i, 0)),
            pl.BlockSpec((16, N_NODES), lambda i: (0, 0)),
            pl.BlockSpec((N_NODES, N_FEAT), lambda i: (0, 0)),
        ],
        out_specs=[
            pl.BlockSpec((SBLK, N_FEAT), lambda i: (i, 0)),
            pl.BlockSpec((SBLK, 8), lambda i: (i, 0)),
        ],
        out_shape=[
            jax.ShapeDtypeStruct((N_NODES, N_FEAT), jnp.float32),
            jax.ShapeDtypeStruct((N_NODES, 8), jnp.float32),
        ],
    )(adj, nl, nlT, h)


def _passB_body(adj_ref, uv_ref, uvT_ref, xo_ref, acc_ref, rs_ref):
    a = adj_ref[...]
    u = uv_ref[:, 0:1]
    v = uvT_ref[1:2, :]
    e2 = jnp.exp(-_lrelu(u + v)) * a
    acc_ref[...] = jnp.dot(e2, xo_ref[...], preferred_element_type=jnp.float32,
                           precision=lax.Precision.HIGHEST)
    rs_ref[...] = jnp.concatenate(
        [jnp.sum(e2, axis=1, keepdims=True),
         jnp.zeros((SBLK, 7), jnp.float32)], axis=1)


def _passB(adj, uv, uvT, xo):
    return pl.pallas_call(
        _passB_body,
        grid=(N_NODES // SBLK,),
        in_specs=[
            pl.BlockSpec((SBLK, N_NODES), lambda i: (i, 0)),
            pl.BlockSpec((SBLK, 8), lambda i: (i, 0)),
            pl.BlockSpec((8, N_NODES), lambda i: (0, 0)),
            pl.BlockSpec((N_NODES, N_CLASS), lambda i: (0, 0)),
        ],
        out_specs=[
            pl.BlockSpec((SBLK, N_CLASS), lambda i: (i, 0)),
            pl.BlockSpec((SBLK, 8), lambda i: (i, 0)),
        ],
        out_shape=[
            jax.ShapeDtypeStruct((N_NODES, N_CLASS), jnp.float32),
            jax.ShapeDtypeStruct((N_NODES, 8), jnp.float32),
        ],
    )(adj, uv, uvT, xo)


def _tc3_body(acc_ref, rs_ref, out_ref):
    xf = _elu(acc_ref[...] / (rs_ref[:, 0:1] + EPS))
    m = jnp.max(xf, axis=1, keepdims=True)
    s = xf - m
    out_ref[...] = s - jnp.log(jnp.sum(jnp.exp(s), axis=1, keepdims=True))


def _tc3(acc2, rs2):
    grid = N_NODES // ROW_BLK
    return pl.pallas_call(
        _tc3_body,
        grid=(grid,),
        in_specs=[
            pl.BlockSpec((ROW_BLK, N_CLASS), lambda i: (i, 0)),
            pl.BlockSpec((ROW_BLK, 8), lambda i: (i, 0)),
        ],
        out_specs=pl.BlockSpec((ROW_BLK, N_CLASS), lambda i: (i, 0)),
        out_shape=jax.ShapeDtypeStruct((N_NODES, N_CLASS), jnp.float32),
    )(acc2, rs2)


def kernel(x, adj, W_ks, a_k_param, W_o, a_att, a_out):
    wc = jnp.transpose(W_ks, (1, 0, 2)).reshape(N_FEAT, N_FACT * N_HID)
    a_s = jnp.transpose(a_k_param[:, :N_FEAT])
    a_d = jnp.transpose(a_k_param[:, N_FEAT:])
    a_p = jnp.zeros((N_FEAT, N_FACT), jnp.float32)
    a_q = jnp.zeros((N_FEAT, N_FACT), jnp.float32)
    for k in range(N_FACT):
        a_p = a_p.at[k * N_HID:(k + 1) * N_HID, k].set(a_att[k, 0, :N_HID])
        a_q = a_q.at[k * N_HID:(k + 1) * N_HID, k].set(a_att[k, 0, N_HID:])
    acat = jnp.concatenate([a_s, a_d, a_p, a_q], axis=1)
    ao = jnp.zeros((N_CLASS, 8), jnp.float32)
    ao = ao.at[:, 0].set(a_out[0, :N_CLASS])
    ao = ao.at[:, 1].set(a_out[0, N_CLASS:])

    h, nl = _tc1(x, wc, acat)
    nlT = jnp.transpose(nl)

    acc, rs_pad = _passA(adj, nl, nlT, h)
    xo, uv, sc = _tc2(acc, rs_pad, W_o, ao)
    att_loss = sc[0, 0] / sc[0, 1]

    uvT = jnp.transpose(uv)
    acc2, rs2_pad = _passB(adj, uv, uvT, xo)
    logp = _tc3(acc2, rs2_pad)
    return (logp, att_loss)


# speedup vs baseline: 16.6320x; 1.8814x over previous
import functools
import jax
import jax.numpy as jnp
from jax import lax
from jax.experimental import pallas as pl

N_NODES = 10000
N_FEAT = 256
N_HID = 64
N_FACT = 4
N_CLASS = 16
N_EDGES = 160000
ALPHA = 0.2
EPS = 1e-16

ROW_BLK = 1000


def _lrelu(z):
    return jnp.maximum(z, ALPHA * z)


def _elu(x):
    return jnp.where(x > 0, x, jnp.exp(jnp.minimum(x, 0.0)) - 1.0)


def _bdot(a, b):
    return jnp.dot(a.astype(jnp.bfloat16), b.astype(jnp.bfloat16),
                   preferred_element_type=jnp.float32)


def _tc1_body(x_ref, wc_ref, acat_ref, h_ref, nl_ref):
    xb = x_ref[...]
    hb = _bdot(xb, wc_ref[...])
    h_ref[...] = hb
    nl_ref[...] = _bdot(hb, acat_ref[...])


def _tc1(x, wc, acat):
    grid = N_NODES // ROW_BLK
    return pl.pallas_call(
        _tc1_body,
        grid=(grid,),
        in_specs=[
            pl.BlockSpec((ROW_BLK, N_FEAT), lambda i: (i, 0)),
            pl.BlockSpec((N_FEAT, N_FEAT), lambda i: (0, 0)),
            pl.BlockSpec((N_FEAT, 16), lambda i: (0, 0)),
        ],
        out_specs=[
            pl.BlockSpec((ROW_BLK, N_FEAT), lambda i: (i, 0)),
            pl.BlockSpec((ROW_BLK, 16), lambda i: (i, 0)),
        ],
        out_shape=[
            jax.ShapeDtypeStruct((N_NODES, N_FEAT), jnp.float32),
            jax.ShapeDtypeStruct((N_NODES, 16), jnp.float32),
        ],
    )(x, wc, acat)


def _tc2_body(acc_ref, rs_ref, wo_ref, ao_ref, xo_ref, uv_ref, sc_ref):
    i = pl.program_id(0)
    acc = acc_ref[...]
    rs = rs_ref[...]
    cols = []
    for k in range(N_FACT):
        hp = acc[:, k * N_HID:(k + 1) * N_HID] / (rs[:, k:k + 1] + EPS)
        cols.append(_elu(hp))
    xc = jnp.concatenate(cols, axis=1)
    xo = _bdot(xc, wo_ref[...])
    xo_ref[...] = xo
    uv_ref[...] = _bdot(xo, ao_ref[...])
    upd = jnp.concatenate(
        [jnp.sum(rs[:, 4:5], axis=0, keepdims=True),
         jnp.sum(rs[:, 5:6], axis=0, keepdims=True),
         jnp.zeros((1, 6), jnp.float32)], axis=1)

    @pl.when(i == 0)
    def _():
        sc_ref[...] = upd

    @pl.when(i != 0)
    def _():
        sc_ref[...] = sc_ref[...] + upd


def _tc2(acc, rs, wo, ao):
    grid = N_NODES // ROW_BLK
    return pl.pallas_call(
        _tc2_body,
        grid=(grid,),
        in_specs=[
            pl.BlockSpec((ROW_BLK, N_FEAT), lambda i: (i, 0)),
            pl.BlockSpec((ROW_BLK, 8), lambda i: (i, 0)),
            pl.BlockSpec((N_FEAT, N_CLASS), lambda i: (0, 0)),
            pl.BlockSpec((N_CLASS, 8), lambda i: (0, 0)),
        ],
        out_specs=[
            pl.BlockSpec((ROW_BLK, N_CLASS), lambda i: (i, 0)),
            pl.BlockSpec((ROW_BLK, 8), lambda i: (i, 0)),
            pl.BlockSpec((1, 8), lambda i: (0, 0)),
        ],
        out_shape=[
            jax.ShapeDtypeStruct((N_NODES, N_CLASS), jnp.float32),
            jax.ShapeDtypeStruct((N_NODES, 8), jnp.float32),
            jax.ShapeDtypeStruct((1, 8), jnp.float32),
        ],
    )(acc, rs, wo, ao)


SBLK = 80


def _passA_body(adj_ref, nls_ref, nlT_ref, h_ref, acc_ref, rs_ref):
    a = adj_ref[...]
    Sn = nls_ref[...]
    Tn = nlT_ref[...]
    l = [Sn[:, k:k + 1] + Tn[4 + k:5 + k, :] for k in range(N_FACT)]
    max01 = jnp.maximum(l[0], l[1])
    max23 = jnp.maximum(l[2], l[3])
    min01 = jnp.minimum(l[0], l[1])
    min23 = jnp.minimum(l[2], l[3])
    m = jnp.maximum(max01, max23)
    m2 = jnp.maximum(jnp.minimum(max01, max23), jnp.maximum(min01, min23))
    masks = [lk >= m2 for lk in l]
    ex = [jnp.exp(lk - m) for lk in l]
    sumex = ex[0] + ex[1] + ex[2] + ex[3]
    topex = sum(jnp.where(mk, ek, 0.0) for mk, ek in zip(masks, ex))
    lossrow = jnp.sum(a * (1.0 - topex / sumex), axis=1, keepdims=True)
    countrow = jnp.sum(a, axis=1, keepdims=True)
    mats = []
    rows = []
    for k in range(N_FACT):
        z = Sn[:, 8 + k:9 + k] + Tn[12 + k:13 + k, :]
        w = jnp.where(masks[k], jnp.exp(-_lrelu(z)), 0.0) * a
        mats.append(_bdot(w, h_ref[:, k * N_HID:(k + 1) * N_HID]))
        rows.append(jnp.sum(w, axis=1, keepdims=True))
    acc_ref[...] = jnp.concatenate(mats, axis=1)
    rs_ref[...] = jnp.concatenate(
        rows + [lossrow, countrow, jnp.zeros((SBLK, 2), jnp.float32)], axis=1)


def _passA(adj, nl, nlT, h):
    return pl.pallas_call(
        _passA_body,
        grid=(N_NODES // SBLK,),
        in_specs=[
            pl.BlockSpec((SBLK, N_NODES), lambda i: (i, 0)),
            pl.BlockSpec((SBLK, 16), lambda i: (i, 0)),
            pl.BlockSpec((16, N_NODES), lambda i: (0, 0)),
            pl.BlockSpec((N_NODES, N_FEAT), lambda i: (0, 0)),
        ],
        out_specs=[
            pl.BlockSpec((SBLK, N_FEAT), lambda i: (i, 0)),
            pl.BlockSpec((SBLK, 8), lambda i: (i, 0)),
        ],
        out_shape=[
            jax.ShapeDtypeStruct((N_NODES, N_FEAT), jnp.float32),
            jax.ShapeDtypeStruct((N_NODES, 8), jnp.float32),
        ],
    )(adj, nl, nlT, h)


def _passB_body(adj_ref, uv_ref, uvT_ref, xo_ref, acc_ref, rs_ref):
    a = adj_ref[...]
    u = uv_ref[:, 0:1]
    v = uvT_ref[1:2, :]
    e2 = jnp.exp(-_lrelu(u + v)) * a
    acc_ref[...] = _bdot(e2, xo_ref[...])
    rs_ref[...] = jnp.concatenate(
        [jnp.sum(e2, axis=1, keepdims=True),
         jnp.zeros((SBLK, 7), jnp.float32)], axis=1)


def _passB(adj, uv, uvT, xo):
    return pl.pallas_call(
        _passB_body,
        grid=(N_NODES // SBLK,),
        in_specs=[
            pl.BlockSpec((SBLK, N_NODES), lambda i: (i, 0)),
            pl.BlockSpec((SBLK, 8), lambda i: (i, 0)),
            pl.BlockSpec((8, N_NODES), lambda i: (0, 0)),
            pl.BlockSpec((N_NODES, N_CLASS), lambda i: (0, 0)),
        ],
        out_specs=[
            pl.BlockSpec((SBLK, N_CLASS), lambda i: (i, 0)),
            pl.BlockSpec((SBLK, 8), lambda i: (i, 0)),
        ],
        out_shape=[
            jax.ShapeDtypeStruct((N_NODES, N_CLASS), jnp.float32),
            jax.ShapeDtypeStruct((N_NODES, 8), jnp.float32),
        ],
    )(adj, uv, uvT, xo)


def _tc3_body(acc_ref, rs_ref, out_ref):
    xf = _elu(acc_ref[...] / (rs_ref[:, 0:1] + EPS))
    m = jnp.max(xf, axis=1, keepdims=True)
    s = xf - m
    out_ref[...] = s - jnp.log(jnp.sum(jnp.exp(s), axis=1, keepdims=True))


def _tc3(acc2, rs2):
    grid = N_NODES // ROW_BLK
    return pl.pallas_call(
        _tc3_body,
        grid=(grid,),
        in_specs=[
            pl.BlockSpec((ROW_BLK, N_CLASS), lambda i: (i, 0)),
            pl.BlockSpec((ROW_BLK, 8), lambda i: (i, 0)),
        ],
        out_specs=pl.BlockSpec((ROW_BLK, N_CLASS), lambda i: (i, 0)),
        out_shape=jax.ShapeDtypeStruct((N_NODES, N_CLASS), jnp.float32),
    )(acc2, rs2)


def kernel(x, adj, W_ks, a_k_param, W_o, a_att, a_out):
    wc = jnp.transpose(W_ks, (1, 0, 2)).reshape(N_FEAT, N_FACT * N_HID)
    a_s = jnp.transpose(a_k_param[:, :N_FEAT])
    a_d = jnp.transpose(a_k_param[:, N_FEAT:])
    a_p = jnp.zeros((N_FEAT, N_FACT), jnp.float32)
    a_q = jnp.zeros((N_FEAT, N_FACT), jnp.float32)
    for k in range(N_FACT):
        a_p = a_p.at[k * N_HID:(k + 1) * N_HID, k].set(a_att[k, 0, :N_HID])
        a_q = a_q.at[k * N_HID:(k + 1) * N_HID, k].set(a_att[k, 0, N_HID:])
    acat = jnp.concatenate([a_s, a_d, a_p, a_q], axis=1)
    ao = jnp.zeros((N_CLASS, 8), jnp.float32)
    ao = ao.at[:, 0].set(a_out[0, :N_CLASS])
    ao = ao.at[:, 1].set(a_out[0, N_CLASS:])

    h, nl = _tc1(x, wc, acat)
    nlT = jnp.transpose(nl)

    acc, rs_pad = _passA(adj, nl, nlT, h)
    xo, uv, sc = _tc2(acc, rs_pad, W_o, ao)
    att_loss = sc[0, 0] / sc[0, 1]

    uvT = jnp.transpose(uv)
    acc2, rs2_pad = _passB(adj, uv, uvT, xo)
    logp = _tc3(acc2, rs2_pad)
    return (logp, att_loss)
